# baseline (speedup 1.0000x reference)
import functools

import jax
import jax.numpy as jnp
from jax import lax
from jax.experimental import pallas as pl
from jax.experimental.pallas import tpu as pltpu
from jax.experimental.pallas import tpu_sc as plsc

NC = 2
NS = 16
NW = NC * NS

B = 4096
H = 200
D = 32
TOTAL = B * H
PER_TILE = TOTAL // NW
CHUNK = 256
NCHUNK = PER_TILE // CHUNK
K = 5
G = NCHUNK // K

_mesh = plsc.VectorSubcoreMesh(
    core_axis_name="c", subcore_axis_name="s", num_cores=NC, num_subcores=NS)


@functools.partial(
    pl.kernel,
    out_type=jax.ShapeDtypeStruct((NW, NCHUNK, CHUNK, D), jnp.float32),
    mesh=_mesh,
    scratch_types=[
        pltpu.VMEM((NCHUNK, CHUNK), jnp.int32),
        pltpu.VMEM((2, K, CHUNK, D), jnp.float32),
        pltpu.SemaphoreType.DMA,
        pltpu.SemaphoreType.DMA,
        pltpu.SemaphoreType.DMA,
    ],
    compiler_params=pltpu.CompilerParams(use_tc_tiling_on_sc=False),
)
def _gather_kernel(idx_hbm, table_hbm, out_hbm, idx_v, rows_v, gsem0, gsem1,
                   osem):
    wid = lax.axis_index("s") * NC + lax.axis_index("c")
    pltpu.sync_copy(idx_hbm.at[wid], idx_v)

    gsems = (gsem0, gsem1)

    def fire_gathers(g, p):
        for b in range(K):
            pltpu.async_copy(
                table_hbm.at[idx_v.at[g * K + b]], rows_v.at[p, b], gsems[p])

    def drain_gathers(p):
        pltpu.make_async_copy(
            out_hbm.at[0, pl.ds(0, K)], rows_v.at[p], gsems[p]).wait()

    def wait_one_out():
        pltpu.make_async_copy(
            rows_v.at[0], out_hbm.at[0, pl.ds(0, K)], osem).wait()

    def step(g, p):
        drain_gathers(p)
        @pl.when(g >= 1)
        def _():
            wait_one_out()

        pltpu.async_copy(rows_v.at[p], out_hbm.at[wid, pl.ds(g * K, K)], osem)

        @pl.when(g + 1 < G)
        def _():
            fire_gathers(g + 1, 1 - p)

    fire_gathers(0, 0)

    def pair(i, carry):
        step(2 * i, 0)
        step(2 * i + 1, 1)
        return carry

    lax.fori_loop(0, G // 2, pair, 0)
    wait_one_out()


def kernel(sequences, sequence_legths, table):
    idx = sequences.reshape(NW, NCHUNK, CHUNK).astype(jnp.int32)
    out = _gather_kernel(idx, table)
    return out.reshape(B, H, D)


# speedup vs baseline: 1.0046x; 1.0046x over previous
import functools

import jax
import jax.numpy as jnp
from jax import lax
from jax.experimental import pallas as pl
from jax.experimental.pallas import tpu as pltpu
from jax.experimental.pallas import tpu_sc as plsc

NC = 2
NS = 16
NW = NC * NS

B = 4096
H = 200
D = 32
TOTAL = B * H
PER_TILE = TOTAL // NW
CHUNK = 256
NCHUNK = PER_TILE // CHUNK
K = 4
G = NCHUNK // K
NSLOT = 3

_mesh = plsc.VectorSubcoreMesh(
    core_axis_name="c", subcore_axis_name="s", num_cores=NC, num_subcores=NS)


@functools.partial(
    pl.kernel,
    out_type=jax.ShapeDtypeStruct((NW, NCHUNK, CHUNK, D), jnp.float32),
    mesh=_mesh,
    scratch_types=[
        pltpu.VMEM((NCHUNK, CHUNK), jnp.int32),
        pltpu.VMEM((NSLOT, K, CHUNK, D), jnp.float32),
        pltpu.SemaphoreType.DMA,
        pltpu.SemaphoreType.DMA,
        pltpu.SemaphoreType.DMA,
        pltpu.SemaphoreType.DMA,
    ],
    compiler_params=pltpu.CompilerParams(use_tc_tiling_on_sc=False),
)
def _gather_kernel(idx_hbm, table_hbm, out_hbm, idx_v, rows_v, gsem0, gsem1,
                   gsem2, osem):
    wid = lax.axis_index("s") * NC + lax.axis_index("c")
    pltpu.sync_copy(idx_hbm.at[wid], idx_v)

    gsems = (gsem0, gsem1, gsem2)

    def fire_gathers(g, s):
        for b in range(K):
            pltpu.async_copy(
                table_hbm.at[idx_v.at[g * K + b]], rows_v.at[s, b], gsems[s])

    def drain_gathers(s):
        pltpu.make_async_copy(
            out_hbm.at[0, pl.ds(0, K)], rows_v.at[s], gsems[s]).wait()

    def wait_one_out():
        pltpu.make_async_copy(
            rows_v.at[0], out_hbm.at[0, pl.ds(0, K)], osem).wait()

    def step(g, s):
        drain_gathers(s)
        @pl.when(g >= 1)
        def _():
            wait_one_out()

        pltpu.async_copy(rows_v.at[s], out_hbm.at[wid, pl.ds(g * K, K)], osem)

        @pl.when(g + 2 < G)
        def _():
            fire_gathers(g + 2, (s + 2) % NSLOT)

    fire_gathers(0, 0)
    fire_gathers(1, 1)

    def triple(i, carry):
        g = 3 * i
        step(g, 0)
        step(g + 1, 1)
        step(g + 2, 2)
        return carry

    lax.fori_loop(0, G // 3, triple, 0)
    step(G - 1, (G - 1) % NSLOT)
    wait_one_out()


def kernel(sequences, sequence_legths, table):
    idx = sequences.reshape(NW, NCHUNK, CHUNK).astype(jnp.int32)
    out = _gather_kernel(idx, table)
    return out.reshape(B, H, D)


# speedup vs baseline: 1.0047x; 1.0001x over previous
import functools

import jax
import jax.numpy as jnp
from jax import lax
from jax.experimental import pallas as pl
from jax.experimental.pallas import tpu as pltpu
from jax.experimental.pallas import tpu_sc as plsc

NC = 2
NS = 16
NW = NC * NS

B = 4096
H = 200
D = 32
TOTAL = B * H
PER_TILE = TOTAL // NW
CHUNK = 128
NCHUNK = PER_TILE // CHUNK
K = 5
G = NCHUNK // K
NSLOT = 4

_mesh = plsc.VectorSubcoreMesh(
    core_axis_name="c", subcore_axis_name="s", num_cores=NC, num_subcores=NS)


@functools.partial(
    pl.kernel,
    out_type=jax.ShapeDtypeStruct((NW, NCHUNK, CHUNK, D), jnp.float32),
    mesh=_mesh,
    scratch_types=[
        pltpu.VMEM((NCHUNK, CHUNK), jnp.int32),
        pltpu.VMEM((NSLOT, K, CHUNK, D), jnp.float32),
        pltpu.SemaphoreType.DMA,
        pltpu.SemaphoreType.DMA,
        pltpu.SemaphoreType.DMA,
        pltpu.SemaphoreType.DMA,
        pltpu.SemaphoreType.DMA,
        pltpu.SemaphoreType.DMA,
    ],
    compiler_params=pltpu.CompilerParams(use_tc_tiling_on_sc=False),
)
def _gather_kernel(idx_hbm, table_hbm, out_hbm, idx_v, rows_v, gsem0, gsem1,
                   gsem2, gsem3, osem0, osem1):
    wid = lax.axis_index("s") * NC + lax.axis_index("c")
    pltpu.sync_copy(idx_hbm.at[wid], idx_v)

    gsems = (gsem0, gsem1, gsem2, gsem3)
    osems = (osem0, osem1)

    def fire_gathers(g, s):
        for b in range(K):
            pltpu.async_copy(
                table_hbm.at[idx_v.at[g * K + b]], rows_v.at[s, b], gsems[s])

    def drain_gathers(s):
        pltpu.make_async_copy(
            out_hbm.at[0, pl.ds(0, K)], rows_v.at[s], gsems[s]).wait()

    def wait_out(q):
        pltpu.make_async_copy(
            rows_v.at[0], out_hbm.at[0, pl.ds(0, K)], osems[q]).wait()

    def step(g, s):
        drain_gathers(s)
        @pl.when(g >= 2)
        def _():
            wait_out(s % 2)

        pltpu.async_copy(
            rows_v.at[s], out_hbm.at[wid, pl.ds(g * K, K)], osems[s % 2])

        @pl.when(g + 2 < G)
        def _():
            fire_gathers(g + 2, (s + 2) % NSLOT)

    fire_gathers(0, 0)
    fire_gathers(1, 1)

    def quad(i, carry):
        g = 4 * i
        step(g, 0)
        step(g + 1, 1)
        step(g + 2, 2)
        step(g + 3, 3)
        return carry

    lax.fori_loop(0, G // 4, quad, 0)
    wait_out(0)
    wait_out(1)


def kernel(sequences, sequence_legths, table):
    idx = sequences.reshape(NW, NCHUNK, CHUNK).astype(jnp.int32)
    out = _gather_kernel(idx, table)
    return out.reshape(B, H, D)


# speedup vs baseline: 1.0047x; 1.0000x over previous
import functools

import jax
import jax.numpy as jnp
from jax import lax
from jax.experimental import pallas as pl
from jax.experimental.pallas import tpu as pltpu
from jax.experimental.pallas import tpu_sc as plsc

NC = 2
NS = 16
NW = NC * NS

B = 4096
H = 200
D = 32
TOTAL = B * H
PER_TILE = TOTAL // NW
CHUNK = 128
NCHUNK = PER_TILE // CHUNK
K = 5
G = NCHUNK // K
NSLOT = 4

_mesh = plsc.VectorSubcoreMesh(
    core_axis_name="c", subcore_axis_name="s", num_cores=NC, num_subcores=NS)


@functools.partial(
    pl.kernel,
    out_type=jax.ShapeDtypeStruct((NW, NCHUNK, CHUNK, D), jnp.float32),
    mesh=_mesh,
    scratch_types=[
        pltpu.VMEM((NCHUNK, CHUNK), jnp.int32),
        pltpu.VMEM((NSLOT, K, CHUNK, D), jnp.float32),
        pltpu.SemaphoreType.DMA,
        pltpu.SemaphoreType.DMA,
        pltpu.SemaphoreType.DMA,
        pltpu.SemaphoreType.DMA,
        pltpu.SemaphoreType.DMA,
        pltpu.SemaphoreType.DMA,
        pltpu.SemaphoreType.DMA,
    ],
    compiler_params=pltpu.CompilerParams(use_tc_tiling_on_sc=False),
)
def _gather_kernel(idx_hbm, table_hbm, out_hbm, idx_v, rows_v, gsem0, gsem1,
                   gsem2, gsem3, osem0, osem1, ssem):
    wid = lax.axis_index("s") * NC + lax.axis_index("c")
    PRE = 2 * K
    pltpu.sync_copy(idx_hbm.at[wid, pl.ds(0, PRE)], idx_v.at[pl.ds(0, PRE)])
    pltpu.async_copy(
        idx_hbm.at[wid, pl.ds(PRE, NCHUNK - PRE)],
        idx_v.at[pl.ds(PRE, NCHUNK - PRE)], ssem)

    gsems = (gsem0, gsem1, gsem2, gsem3)
    osems = (osem0, osem1)

    def fire_gathers(g, s):
        for b in range(K):
            pltpu.async_copy(
                table_hbm.at[idx_v.at[g * K + b]], rows_v.at[s, b], gsems[s])

    def drain_gathers(s):
        pltpu.make_async_copy(
            out_hbm.at[0, pl.ds(0, K)], rows_v.at[s], gsems[s]).wait()

    def wait_out(q):
        pltpu.make_async_copy(
            rows_v.at[0], out_hbm.at[0, pl.ds(0, K)], osems[q]).wait()

    def step(g, s):
        drain_gathers(s)
        @pl.when(g >= 2)
        def _():
            wait_out(s % 2)

        pltpu.async_copy(
            rows_v.at[s], out_hbm.at[wid, pl.ds(g * K, K)], osems[s % 2])

        @pl.when(g + 2 < G)
        def _():
            fire_gathers(g + 2, (s + 2) % NSLOT)

    fire_gathers(0, 0)
    fire_gathers(1, 1)
    pltpu.make_async_copy(
        idx_hbm.at[wid, pl.ds(PRE, NCHUNK - PRE)],
        idx_v.at[pl.ds(PRE, NCHUNK - PRE)], ssem).wait()

    def quad(i, carry):
        g = 4 * i
        step(g, 0)
        step(g + 1, 1)
        step(g + 2, 2)
        step(g + 3, 3)
        return carry

    lax.fori_loop(0, G // 4, quad, 0)
    wait_out(0)
    wait_out(1)


def kernel(sequences, sequence_legths, table):
    idx = sequences.reshape(NW, NCHUNK, CHUNK).astype(jnp.int32)
    out = _gather_kernel(idx, table)
    return out.reshape(B, H, D)
